# Initial kernel scaffold; baseline (speedup 1.0000x reference)
import jax
import jax.numpy as jnp
from jax.experimental import pallas as pl


def kernel(hidden_states, wg, fc1_1, fc1_2, fc2):
    raise NotImplementedError("write your pallas kernel here")


# speedup vs baseline: 1.7872x; 1.7872x over previous
import jax
import jax.numpy as jnp
from jax.experimental import pallas as pl
from jax.experimental.pallas import tpu as pltpu

E = 8
D = 1024
F = 2816
FT = 256
NF = F // FT

_PREC = jax.lax.Precision.DEFAULT


def _router_kernel(x_ref, wg_ref, logits_ref, w_ref):
    x = x_ref[...]
    wg = wg_ref[...] * 0.5
    logits = jax.lax.dot_general(
        x, wg, (((1,), (0,)), ((), ())),
        preferred_element_type=jnp.float32,
        precision=jax.lax.Precision.DEFAULT)
    logits_ref[...] = logits
    m = jnp.max(logits, axis=1, keepdims=True)
    ex = jnp.exp(logits - m)
    probs = ex / jnp.sum(ex, axis=1, keepdims=True)
    idx = jax.lax.broadcasted_iota(jnp.int32, probs.shape, 1)
    m1 = jnp.max(probs, axis=1, keepdims=True)
    i1 = jnp.min(jnp.where(probs == m1, idx, E), axis=1, keepdims=True)
    pm = jnp.where(idx == i1, -jnp.inf, probs)
    m2 = jnp.max(pm, axis=1, keepdims=True)
    i2 = jnp.min(jnp.where(pm == m2, idx, E), axis=1, keepdims=True)
    sel = (idx == i1) | (idx == i2)
    w_ref[...] = jnp.where(sel, probs, 0.0) / (m1 + m2)


def _moe_kernel(x_ref, w1_ref, w2_ref, w3_ref, w_ref, out_ref):
    e = pl.program_id(0)
    f = pl.program_id(1)

    @pl.when((e == 0) & (f == 0))
    def _():
        out_ref[...] = jnp.zeros_like(out_ref)

    x = x_ref[...]
    h1 = jax.lax.dot_general(
        x, w1_ref[0], (((1,), (1,)), ((), ())),
        preferred_element_type=jnp.float32, precision=_PREC)
    h2 = jax.lax.dot_general(
        x, w2_ref[0], (((1,), (1,)), ((), ())),
        preferred_element_type=jnp.float32, precision=_PREC)
    h = (h1 * jax.nn.sigmoid(h1)) * h2
    o = jax.lax.dot_general(
        h.astype(x.dtype), w3_ref[0], (((1,), (1,)), ((), ())),
        preferred_element_type=jnp.float32, precision=_PREC)
    eidx = jax.lax.broadcasted_iota(jnp.int32, w_ref.shape, 1)
    wcol = jnp.sum(jnp.where(eidx == e, w_ref[...], 0.0), axis=1, keepdims=True)
    out_ref[...] += o * wcol


def kernel(hidden_states, wg, fc1_1, fc1_2, fc2):
    B, S, _ = hidden_states.shape
    T = B * S
    x = hidden_states.reshape(T, D)

    logits, w = pl.pallas_call(
        _router_kernel,
        out_shape=(
            jax.ShapeDtypeStruct((T, E), jnp.float32),
            jax.ShapeDtypeStruct((T, E), jnp.float32),
        ),
    )(x, wg)

    final = pl.pallas_call(
        _moe_kernel,
        grid=(E, NF),
        in_specs=[
            pl.BlockSpec((T, D), lambda e, f: (0, 0)),
            pl.BlockSpec((1, FT, D), lambda e, f: (e, f, 0)),
            pl.BlockSpec((1, FT, D), lambda e, f: (e, f, 0)),
            pl.BlockSpec((1, D, FT), lambda e, f: (e, 0, f)),
            pl.BlockSpec((T, E), lambda e, f: (0, 0)),
        ],
        out_specs=pl.BlockSpec((T, D), lambda e, f: (0, 0)),
        out_shape=jax.ShapeDtypeStruct((T, D), jnp.float32),
    )(x, fc1_1, fc1_2, fc2, w)

    return final.reshape(B, S, D), logits
